# baseline (speedup 1.0000x reference)
import functools

import jax
import jax.numpy as jnp
from jax.experimental import pallas as pl
from jax.experimental.pallas import tpu as pltpu


def _tree_min(chunks):
    while len(chunks) > 1:
        nxt = []
        for i in range(0, len(chunks) - 1, 2):
            nxt.append(jnp.minimum(chunks[i], chunks[i + 1]))
        if len(chunks) % 2:
            nxt.append(chunks[-1])
        chunks = nxt
    return chunks[0]


def _chamfer_body(
    a_ref, bt_ref, xsq_ref, ysq_ref, acc_ref, row_s, col_s, *, bn, bm, ck, inv_bn, inv_bm
):
    n = pl.program_id(1)
    m = pl.program_id(2)

    a = a_ref[0]
    xsqc = xsq_ref[0]

    rowacc = None
    colaccs = []

    for k in range(0, bm, ck):
        inner = jnp.dot(
            a, bt_ref[0, :, k : k + ck], preferred_element_type=jnp.float32
        )
        ysqk = ysq_ref[0, :, k : k + ck]
        e = inner + ysqk
        f = inner + xsqc

        rowpart = _tree_min([e[:, j : j + 128] for j in range(0, ck, 128)])
        rowacc = rowpart if rowacc is None else jnp.minimum(rowacc, rowpart)

        colaccs.append(_tree_min([f[j : j + 8, :] for j in range(0, bn, 8)]))

    prev1 = jnp.where(m == 0, jnp.inf, row_s[...])
    row_s[...] = jnp.minimum(prev1, rowacc)

    colacc = jnp.concatenate(colaccs, axis=1)
    prev2 = jnp.where(n == 0, jnp.inf, col_s[m])
    col_s[m] = jnp.minimum(prev2, colacc)

    @pl.when((pl.program_id(0) == 0) & (n == 0) & (m == 0))
    def _init():
        acc_ref[0, 0] = 0.0

    @pl.when(m == pl.num_programs(2) - 1)
    def _fin1():
        dist1 = jnp.min(row_s[...], axis=1, keepdims=True) + xsqc
        acc_ref[0, 0] += jnp.sum(dist1) * inv_bn

    @pl.when(n == pl.num_programs(1) - 1)
    def _fin2():
        dist2 = jnp.min(col_s[m], axis=0) + ysq_ref[0, 0, :]
        acc_ref[0, 0] += jnp.sum(dist2) * inv_bm


@jax.jit
def kernel(xyz1, xyz2):
    B, N, _ = xyz1.shape
    M = xyz2.shape[1]
    BN = 256
    BM = 2048
    CK = 128
    NB = N // BN
    MB = M // BM

    xsq = jnp.sum(xyz1 * xyz1, axis=-1, keepdims=True)
    ysq = jnp.sum(xyz2 * xyz2, axis=-1)[:, None, :]
    pad_x = jnp.zeros((B, N, 5), jnp.bfloat16)
    pad_y = jnp.zeros((B, M, 5), jnp.bfloat16)
    xb = jnp.concatenate([xyz1.astype(jnp.bfloat16), pad_x], axis=-1)
    yb = jnp.concatenate([-2.0 * xyz2.astype(jnp.bfloat16), pad_y], axis=-1)
    ybt = jnp.transpose(yb, (0, 2, 1))

    body = functools.partial(
        _chamfer_body,
        bn=BN, bm=BM, ck=CK, inv_bn=1.0 / (B * N), inv_bm=1.0 / (B * M),
    )
    acc = pl.pallas_call(
        body,
        grid=(B, NB, MB),
        in_specs=[
            pl.BlockSpec((1, BN, 8), lambda b, n, m: (b, n, 0)),
            pl.BlockSpec((1, 8, BM), lambda b, n, m: (b, 0, m)),
            pl.BlockSpec((1, BN, 1), lambda b, n, m: (b, n, 0)),
            pl.BlockSpec((1, 1, BM), lambda b, n, m: (b, 0, m)),
        ],
        out_specs=pl.BlockSpec(
            (1, 1), lambda b, n, m: (0, 0), memory_space=pltpu.SMEM
        ),
        out_shape=jax.ShapeDtypeStruct((1, 1), jnp.float32),
        scratch_shapes=[
            pltpu.VMEM((BN, 128), jnp.float32),
            pltpu.VMEM((MB, 8, BM), jnp.float32),
        ],
    )(xb, ybt, xsq, ysq)
    return acc[0, 0]


# speedup vs baseline: 1.4833x; 1.4833x over previous
import functools

import jax
import jax.numpy as jnp
from jax.experimental import pallas as pl
from jax.experimental.pallas import tpu as pltpu


def _tree_min(chunks):
    while len(chunks) > 1:
        nxt = []
        for i in range(0, len(chunks) - 1, 2):
            nxt.append(jnp.minimum(chunks[i], chunks[i + 1]))
        if len(chunks) % 2:
            nxt.append(chunks[-1])
        chunks = nxt
    return chunks[0]


def _chamfer_body(x_ref, yt_ref, acc_ref, col_s, *, bn, m_tot, inv_bn, inv_bm):
    n = pl.program_id(1)

    x = x_ref[0]
    yt = yt_ref[0]
    xb = x.astype(jnp.bfloat16)
    ytb = (-2.0 * yt).astype(jnp.bfloat16)
    xsq = jnp.sum(x * x, axis=1, keepdims=True)
    ysq = jnp.sum(yt * yt, axis=0, keepdims=True)

    first_n = n == 0
    rowacc = None
    for k in range(0, m_tot, 128):
        inner = jnp.dot(
            xb, ytb[:, k : k + 128], preferred_element_type=jnp.float32
        )
        e = inner + ysq[:, k : k + 128]
        rowacc = e if rowacc is None else jnp.minimum(rowacc, e)

        f = inner + xsq
        colpart = _tree_min([f[j : j + 8, :] for j in range(0, bn, 8)])
        prev = jnp.where(first_n, jnp.inf, col_s[:, k : k + 128])
        col_s[:, k : k + 128] = jnp.minimum(prev, colpart)

    dist1 = jnp.min(rowacc, axis=1, keepdims=True) + xsq

    @pl.when((pl.program_id(0) == 0) & first_n)
    def _init():
        acc_ref[0, 0] = 0.0

    acc_ref[0, 0] += jnp.sum(dist1) * inv_bn

    @pl.when(n == pl.num_programs(1) - 1)
    def _fin2():
        dist2 = jnp.min(col_s[...], axis=0, keepdims=True) + ysq
        acc_ref[0, 0] += jnp.sum(dist2) * inv_bm


@jax.jit
def kernel(xyz1, xyz2):
    B, N, _ = xyz1.shape
    M = xyz2.shape[1]
    BN = 256
    NB = N // BN

    yt = jnp.transpose(xyz2, (0, 2, 1))

    body = functools.partial(
        _chamfer_body, bn=BN, m_tot=M, inv_bn=1.0 / (B * N), inv_bm=1.0 / (B * M)
    )
    acc = pl.pallas_call(
        body,
        grid=(B, NB),
        in_specs=[
            pl.BlockSpec((1, BN, 3), lambda b, n: (b, n, 0)),
            pl.BlockSpec((1, 3, M), lambda b, n: (b, 0, 0)),
        ],
        out_specs=pl.BlockSpec((1, 1), lambda b, n: (0, 0), memory_space=pltpu.SMEM),
        out_shape=jax.ShapeDtypeStruct((1, 1), jnp.float32),
        scratch_shapes=[
            pltpu.VMEM((8, M), jnp.float32),
        ],
    )(xyz1, yt)
    return acc[0, 0]
